# baseline (speedup 1.0000x reference)
import functools

import jax
import jax.numpy as jnp
from jax import lax
from jax.experimental import pallas as pl
from jax.experimental.pallas import tpu as pltpu
from jax.experimental.pallas import tpu_sc as plsc

_CHUNK = 128


def _z(_):
    return jnp.int32(0)


def _gather_body(chunks_per_tile, table_hbm, idx_hbm, out_hbm,
                 idx_v, rows_a, rows_b, sem_a, sem_b):
    info = plsc.get_sparse_core_info()
    nc = info.num_cores
    wid = lax.axis_index("s") * jnp.int32(nc) + lax.axis_index("c")
    tpt = chunks_per_tile * _CHUNK
    base = wid * jnp.int32(tpt)

    pltpu.sync_copy(idx_hbm.at[pl.ds(base, tpt)], idx_v)
    bufs = [rows_a, rows_b]
    sems = [sem_a, sem_b]

    def start(c):
        return pltpu.async_copy(
            table_hbm.at[idx_v.at[pl.ds(jnp.int32(c * _CHUNK), _CHUNK)]],
            bufs[c % 2], sems[c % 2])

    copies = [start(0), None]
    for c in range(chunks_per_tile):
        nxt = c + 1
        if nxt < chunks_per_tile:
            copies[nxt % 2] = start(nxt)
        copies[c % 2].wait()
        pltpu.sync_copy(
            bufs[c % 2],
            out_hbm.at[pl.ds(base + jnp.int32(c * _CHUNK), _CHUNK)])


def _unpack_bf16_pair(container_f32):
    xi = lax.bitcast_convert_type(container_f32, jnp.int32)
    lo = lax.bitcast_convert_type(xi << jnp.int32(16), jnp.float32)
    hi = lax.bitcast_convert_type(xi & jnp.int32(-65536), jnp.float32)
    return lo.astype(jnp.bfloat16), hi.astype(jnp.bfloat16)


def _fused_body(e2_ref, e3_ref, h_ref, w_ref, gw_ref, o_ref):
    e2_lo, e2_hi = _unpack_bf16_pair(e2_ref[...])
    e3_lo, e3_hi = _unpack_bf16_pair(e3_ref[...])
    v = jnp.dot(e2_lo, w_ref[0], preferred_element_type=jnp.float32)
    v = v + jnp.dot(e2_hi, w_ref[1], preferred_element_type=jnp.float32)
    v = v + jnp.dot(e3_lo, w_ref[2], preferred_element_type=jnp.float32)
    v = v + jnp.dot(e3_hi, w_ref[3], preferred_element_type=jnp.float32)
    h = h_ref[...]
    hid = h.shape[-1]
    sh = jnp.mean(h * h, axis=-1, keepdims=True)
    sv = jnp.mean(v * v, axis=-1, keepdims=True)
    num = jnp.sum(h * v * gw_ref[...], axis=-1, keepdims=True)
    gate = (num * lax.rsqrt(sh + 1e-6) * lax.rsqrt(sv + 1e-6)
            / (hid ** 0.5))
    gate = jnp.sqrt(jnp.maximum(jnp.abs(gate), 1e-6)) * jnp.sign(gate)
    gate = jax.nn.sigmoid(gate)
    o_ref[...] = gate * v


def kernel(hidden, input_ids, compress_table, hash_mult, tables_2gram,
           tables_3gram, W_v, gate_w_h, gate_w_v):
    b, t, hid = hidden.shape
    nh, ts, ed = tables_2gram.shape
    bt = b * t
    dg = nh * ed

    clamped = jnp.clip(input_ids.astype(jnp.int64), 0,
                       compress_table.shape[0] - 1)
    ids = jnp.take(compress_table, clamped, axis=0)
    shifted_1 = jnp.pad(ids[:, :-1], ((0, 0), (1, 0)), constant_values=0)
    shifted_2 = jnp.pad(ids[:, :-2], ((0, 0), (2, 0)), constant_values=0)
    hash_2 = jnp.bitwise_xor(ids * hash_mult[0], shifted_1 * hash_mult[1])
    hash_3 = jnp.bitwise_xor(hash_2, shifted_2 * hash_mult[2])
    idx2 = jnp.maximum(hash_2 % ts, 0).astype(jnp.int32).reshape(-1)
    idx3 = jnp.maximum(hash_3 % ts, 0).astype(jnp.int32).reshape(-1)

    def repack(tab):
        xi = lax.bitcast_convert_type(tab, jnp.int32)
        lsb = lax.shift_right_logical(xi, jnp.int32(16)) & jnp.int32(1)
        rne = lax.shift_right_logical(xi + jnp.int32(0x7FFF) + lsb,
                                      jnp.int32(16))
        cont = rne[0::2] | (rne[1::2] << jnp.int32(16))
        cont = lax.bitcast_convert_type(cont, jnp.float32)
        cont = cont.reshape(nh // 2, ts // 2, 2 * ed)
        cont = lax.optimization_barrier(cont)
        cont = cont.reshape(nh // 2, ts // 2, 2, ed) \
            .transpose(1, 2, 0, 3)
        return cont.reshape(ts, dg // 2)

    t2 = repack(tables_2gram)
    t3 = repack(tables_3gram)

    info = plsc.get_sparse_core_info()
    n_tiles = info.num_cores * info.num_subcores
    chunks_per_tile = bt // (n_tiles * _CHUNK)
    mesh = plsc.VectorSubcoreMesh(core_axis_name="c", subcore_axis_name="s")

    def gather(table, idx):
        return pl.kernel(
            functools.partial(_gather_body, chunks_per_tile),
            mesh=mesh,
            out_type=jax.ShapeDtypeStruct((bt, dg // 2), jnp.float32),
            scratch_types=[
                pltpu.VMEM((bt // n_tiles,), jnp.int32),
                pltpu.VMEM((_CHUNK, dg // 2), jnp.float32),
                pltpu.VMEM((_CHUNK, dg // 2), jnp.float32),
                pltpu.SemaphoreType.DMA,
                pltpu.SemaphoreType.DMA,
            ],
        )(table, idx)

    e2 = gather(t2, idx2)
    e3 = gather(t3, idx3)

    w_t = W_v.T.astype(jnp.bfloat16) \
        .reshape(2, nh // 2, 2, ed, hid).transpose(0, 2, 1, 3, 4) \
        .reshape(4, dg // 2, hid)
    gw = (gate_w_h * gate_w_v).reshape(1, hid)
    blk = 1024
    out = pl.pallas_call(
        _fused_body,
        grid=(bt // blk,),
        in_specs=[
            pl.BlockSpec((blk, dg // 2), lambda i: (i, _z(i))),
            pl.BlockSpec((blk, dg // 2), lambda i: (i, _z(i))),
            pl.BlockSpec((blk, hid), lambda i: (i, _z(i))),
            pl.BlockSpec((4, dg // 2, hid), lambda i: (_z(i), _z(i), _z(i))),
            pl.BlockSpec((1, hid), lambda i: (_z(i), _z(i))),
        ],
        out_specs=pl.BlockSpec((blk, hid), lambda i: (i, _z(i))),
        out_shape=jax.ShapeDtypeStruct((bt, hid), jnp.float32),
    )(e2, e3, hidden.reshape(bt, hid), w_t, gw)

    return out.reshape(b, t, hid)


# speedup vs baseline: 2.3348x; 2.3348x over previous
import functools

import jax
import jax.numpy as jnp
from jax import lax
from jax.experimental import pallas as pl
from jax.experimental.pallas import tpu as pltpu
from jax.experimental.pallas import tpu_sc as plsc

_CHUNK = 128


def _z(_):
    return jnp.int32(0)


def _gather_body(chunks_per_tile, table_hbm, idx_hbm, out_hbm,
                 idx_v, rows_a, rows_b, sem_a, sem_b):
    info = plsc.get_sparse_core_info()
    nc = info.num_cores
    wid = lax.axis_index("s") * jnp.int32(nc) + lax.axis_index("c")
    tpt = chunks_per_tile * _CHUNK
    base = wid * jnp.int32(tpt)

    pltpu.sync_copy(idx_hbm.at[pl.ds(base, tpt)], idx_v)
    bufs = [rows_a, rows_b]
    sems = [sem_a, sem_b]

    def start(c):
        return pltpu.async_copy(
            table_hbm.at[idx_v.at[pl.ds(jnp.int32(c * _CHUNK), _CHUNK)]],
            bufs[c % 2], sems[c % 2])

    copies = [start(0), None]
    for c in range(chunks_per_tile):
        nxt = c + 1
        if nxt < chunks_per_tile:
            copies[nxt % 2] = start(nxt)
        copies[c % 2].wait()
        pltpu.sync_copy(
            bufs[c % 2],
            out_hbm.at[pl.ds(base + jnp.int32(c * _CHUNK), _CHUNK)])


def _fused_body(e2_ref, e3_ref, h_ref, w2_ref, w3_ref, gw_ref, o_ref):
    v = jnp.dot(e2_ref[...].astype(jnp.bfloat16), w2_ref[...],
                preferred_element_type=jnp.float32)
    v = v + jnp.dot(e3_ref[...].astype(jnp.bfloat16), w3_ref[...],
                    preferred_element_type=jnp.float32)
    h = h_ref[...]
    hid = h.shape[-1]
    sh = jnp.mean(h * h, axis=-1, keepdims=True)
    sv = jnp.mean(v * v, axis=-1, keepdims=True)
    num = jnp.sum(h * v * gw_ref[...], axis=-1, keepdims=True)
    gate = (num * lax.rsqrt(sh + 1e-6) * lax.rsqrt(sv + 1e-6)
            / (hid ** 0.5))
    gate = jnp.sqrt(jnp.maximum(jnp.abs(gate), 1e-6)) * jnp.sign(gate)
    gate = jax.nn.sigmoid(gate)
    o_ref[...] = gate * v


def kernel(hidden, input_ids, compress_table, hash_mult, tables_2gram,
           tables_3gram, W_v, gate_w_h, gate_w_v):
    b, t, hid = hidden.shape
    nh, ts, ed = tables_2gram.shape
    bt = b * t
    dg = nh * ed

    clamped = jnp.clip(input_ids.astype(jnp.int64), 0,
                       compress_table.shape[0] - 1)
    ids = jnp.take(compress_table, clamped, axis=0)
    shifted_1 = jnp.pad(ids[:, :-1], ((0, 0), (1, 0)), constant_values=0)
    shifted_2 = jnp.pad(ids[:, :-2], ((0, 0), (2, 0)), constant_values=0)
    hash_2 = jnp.bitwise_xor(ids * hash_mult[0], shifted_1 * hash_mult[1])
    hash_3 = jnp.bitwise_xor(hash_2, shifted_2 * hash_mult[2])
    idx2 = jnp.maximum(hash_2 % ts, 0).astype(jnp.int32).reshape(-1)
    idx3 = jnp.maximum(hash_3 % ts, 0).astype(jnp.int32).reshape(-1)

    t2 = jnp.swapaxes(tables_2gram, 0, 1).reshape(ts, dg)
    t3 = jnp.swapaxes(tables_3gram, 0, 1).reshape(ts, dg)

    info = plsc.get_sparse_core_info()
    n_tiles = info.num_cores * info.num_subcores
    chunks_per_tile = bt // (n_tiles * _CHUNK)
    mesh = plsc.VectorSubcoreMesh(core_axis_name="c", subcore_axis_name="s")

    def gather(table, idx):
        return pl.kernel(
            functools.partial(_gather_body, chunks_per_tile),
            mesh=mesh,
            out_type=jax.ShapeDtypeStruct((bt, dg), jnp.float32),
            scratch_types=[
                pltpu.VMEM((bt // n_tiles,), jnp.int32),
                pltpu.VMEM((_CHUNK, dg), jnp.float32),
                pltpu.VMEM((_CHUNK, dg), jnp.float32),
                pltpu.SemaphoreType.DMA,
                pltpu.SemaphoreType.DMA,
            ],
        )(table, idx)

    e2 = gather(t2, idx2)
    e3 = gather(t3, idx3)

    w_t = W_v.T.astype(jnp.bfloat16)
    gw = (gate_w_h * gate_w_v).reshape(1, hid)
    blk = 1024
    out = pl.pallas_call(
        _fused_body,
        grid=(bt // blk,),
        in_specs=[
            pl.BlockSpec((blk, dg), lambda i: (i, _z(i))),
            pl.BlockSpec((blk, dg), lambda i: (i, _z(i))),
            pl.BlockSpec((blk, hid), lambda i: (i, _z(i))),
            pl.BlockSpec((dg, hid), lambda i: (_z(i), _z(i))),
            pl.BlockSpec((dg, hid), lambda i: (_z(i), _z(i))),
            pl.BlockSpec((1, hid), lambda i: (_z(i), _z(i))),
        ],
        out_specs=pl.BlockSpec((blk, hid), lambda i: (i, _z(i))),
        out_shape=jax.ShapeDtypeStruct((bt, hid), jnp.float32),
    )(e2, e3, hidden.reshape(bt, hid), w_t[:dg], w_t[dg:], gw)

    return out.reshape(b, t, hid)
